# baseline (speedup 1.0000x reference)
import functools

import jax
import jax.numpy as jnp
from jax import lax
from jax.experimental import pallas as pl
from jax.experimental.pallas import tpu as pltpu
from jax.experimental.pallas import tpu_sc as plsc

BATCH = 16384
DIM = 32
NUM_CORES = 2
NUM_SUBCORES = 16
NUM_WORKERS = NUM_CORES * NUM_SUBCORES
B_PER_W = BATCH // NUM_WORKERS
IDX_CHUNK = 128
N_CHUNKS = B_PER_W // IDX_CHUNK
GROUPS = B_PER_W // 16


@functools.partial(
    pl.kernel,
    out_type=jax.ShapeDtypeStruct((BATCH,), jnp.float32),
    mesh=plsc.VectorSubcoreMesh(core_axis_name="c", subcore_axis_name="s"),
    compiler_params=pltpu.CompilerParams(
        needs_layout_passes=False, use_tc_tiling_on_sc=False),
    scratch_types=[
        pltpu.VMEM((N_CHUNKS, IDX_CHUNK), jnp.int32),
        pltpu.VMEM((N_CHUNKS, IDX_CHUNK), jnp.int32),
        pltpu.VMEM((B_PER_W, DIM), jnp.float32),
        pltpu.VMEM((B_PER_W, DIM), jnp.float32),
        pltpu.VMEM((256,), jnp.float32),
        pltpu.VMEM((B_PER_W,), jnp.float32),
        pltpu.SemaphoreType.DMA,
        pltpu.SemaphoreType.DMA,
    ],
)
def _mf_dot_sc(user_hbm, item_hbm, utab_hbm, itab_hbm, out_hbm,
               uidx_v, iidx_v, urows_v, irows_v, part_v, out_v, usem, isem):
    wid = lax.axis_index("s") * NUM_CORES + lax.axis_index("c")
    base = wid * B_PER_W

    for k in range(N_CHUNKS):
        pltpu.sync_copy(user_hbm.at[pl.ds(base + k * IDX_CHUNK, IDX_CHUNK)],
                        uidx_v.at[k])
        pltpu.sync_copy(item_hbm.at[pl.ds(base + k * IDX_CHUNK, IDX_CHUNK)],
                        iidx_v.at[k])

    ucopies = []
    icopies = []
    for k in range(N_CHUNKS):
        ucopies.append(pltpu.async_copy(
            utab_hbm.at[uidx_v.at[k]],
            urows_v.at[pl.ds(k * IDX_CHUNK, IDX_CHUNK)], usem))
        icopies.append(pltpu.async_copy(
            itab_hbm.at[iidx_v.at[k]],
            irows_v.at[pl.ds(k * IDX_CHUNK, IDX_CHUNK)], isem))
    for c in ucopies + icopies:
        c.wait()

    iota = lax.broadcasted_iota(jnp.int32, (16,), 0)

    def group_body(g, carry):
        for rr in range(16):
            r = g * 16 + rr
            p = (urows_v[r, pl.ds(0, 16)] * irows_v[r, pl.ds(0, 16)]
                 + urows_v[r, pl.ds(16, 16)] * irows_v[r, pl.ds(16, 16)])
            part_v[pl.ds(rr * 16, 16)] = p
        acc = jnp.zeros((16,), jnp.float32)
        for j in range(16):
            idx = iota * 16 + ((iota + j) & 15)
            acc = acc + plsc.load_gather(part_v, [idx])
        out_v[pl.ds(g * 16, 16)] = acc
        return carry

    lax.fori_loop(0, GROUPS, group_body, 0)

    pltpu.sync_copy(out_v, out_hbm.at[pl.ds(base, B_PER_W)])


def kernel(user, item, user_emb_table, item_emb_table):
    return _mf_dot_sc(user.astype(jnp.int32), item.astype(jnp.int32),
                      user_emb_table, item_emb_table)
